# baseline (speedup 1.0000x reference)
import functools

import jax
import jax.numpy as jnp
from jax import lax
from jax.experimental import pallas as pl
from jax.experimental.pallas import tpu as pltpu
from jax.experimental.pallas import tpu_sc as plsc

N = 10000
P = 320000
D = 128
CB = 32
NB = D // CB
SEG = 3 * CB
ROW = 4 * CB

NTILES = 16
ROWS_A = 632
ROWS_LAST = N - 15 * ROWS_A
PAIRS_PER_TILE = P // NTILES
WW = 48
NWIN = 418
P_PAD = 15 * PAIRS_PER_TILE + NWIN * WW

TCB = 1000
PCB = 2000


GROW = 256


def _mlp_body(s_ref, v_ref, w1_ref, b1_ref, w2_ref, b2_ref,
              g_ref, res_ref):
    x = s_ref[...]
    h = jax.nn.silu(jnp.dot(x, w1_ref[...], preferred_element_type=jnp.float32)
                    + b1_ref[...])
    t = jnp.dot(h, w2_ref[...], preferred_element_type=jnp.float32) + b2_ref[...]
    vv = v_ref[...]
    pad = jnp.zeros((x.shape[0], GROW - 2 * SEG), jnp.float32)
    for b in range(NB):
        c0 = CB * b
        tb = jnp.concatenate(
            [t[:, k * D + c0:k * D + c0 + CB] for k in range(3)], axis=1)
        vb = jnp.concatenate(
            [vv[:, k * D + c0:k * D + c0 + CB] for k in range(3)], axis=1)
        g_ref[b] = jnp.concatenate([tb, vb, pad], axis=1)
        res_ref[b] = jnp.concatenate([x[:, c0:c0 + CB], vb], axis=1)


def _wblk_body(w_ref, wb_ref):
    w = w_ref[...]
    for b in range(NB):
        c0 = CB * b
        wb_ref[b] = jnp.concatenate(
            [w[:, k * D + c0:k * D + c0 + CB] for k in range(3)], axis=1)


def _sc_body(g_hbm, res_hbm, w_hbm, dir_hbm, ii_hbm, ij_hbm,
             out_hbm,
             acc_sh, idxi0, idxi1, idxj0, idxj1, idxs0, idxs1,
             w0, w1, g0, g1, dir0, dir1, msg0, msg1,
             semi0, semi1, semg0, semg1):
    cid = lax.axis_index("c")
    sid = lax.axis_index("s")
    pair0 = sid * PAIRS_PER_TILE
    idxi_v = (idxi0, idxi1)
    idxj_v = (idxj0, idxj1)
    idxs_v = (idxs0, idxs1)
    w_v = (w0, w1)
    g_v = (g0, g1)
    dir_v = (dir0, dir1)
    msg_v = (msg0, msg1)
    semi = (semi0, semi1)
    semg = (semg0, semg1)

    def rows_io(flush, b):
        def do(row0, nrows):
            src = (acc_sh.at[pl.ds(row0, nrows)] if flush else
                   res_hbm.at[pl.ds(b * N + row0, nrows)])
            dst = (out_hbm.at[pl.ds(b * N + row0, nrows)] if flush else
                   acc_sh.at[pl.ds(row0, nrows)])
            pltpu.sync_copy(src, dst)
        @pl.when(sid < 15)
        def _():
            do(sid * ROWS_A, ROWS_A)
        @pl.when(sid == 15)
        def _():
            do(15 * ROWS_A, ROWS_LAST)

    for bi in range(NB // 2):
        b = cid * (NB // 2) + bi
        rows_io(False, b)
        plsc.subcore_barrier()

        def in_copies(w, rb):
            base = pair0 + w * WW
            return [
                (ii_hbm.at[pl.ds(base, WW)], idxi_v[rb]),
                (ij_hbm.at[pl.ds(base, WW)], idxj_v[rb]),
                (dir_hbm.at[pl.ds(base * 3, WW * 3)],
                 dir_v[rb].at[pl.ds(0, WW * 3)]),
                (w_hbm.at[pl.ds(b * P_PAD + base, WW)], w_v[rb]),
            ]

        def issue_in(w, rb):
            for s, d in in_copies(w, rb):
                pltpu.async_copy(s, d, semi[rb])

        def drain_in(w, rb):
            for s, d in in_copies(w, rb):
                pltpu.make_async_copy(s, d, semi[rb]).wait()

        def shift(rb):
            for i in range(WW // 16):
                idxs_v[rb][pl.ds(i * 16, 16)] = (
                    idxj_v[rb][pl.ds(i * 16, 16)] + b * N)

        def issue_g(rb):
            pltpu.async_copy(g_hbm.at[idxs_v[rb]], g_v[rb], semg[rb])

        def drain_g(rb):
            pltpu.make_async_copy(g_hbm.at[idxs_v[rb]], g_v[rb],
                                  semg[rb]).wait()

        def compute(rb):
            wr, gr, dr, mr = w_v[rb], g_v[rb], dir_v[rb], msg_v[rb]

            def _body(p2, _):
                for u in range(2):
                    p = p2 * 2 + u
                    dvec = dr[pl.ds(p * 3, 16)]
                    dk = [dvec[0], dvec[1], dvec[2]]
                    for j in range(CB // 16):
                        cj = j * 16
                        q = wr[p, pl.ds(cj, 16)] * gr[p, pl.ds(cj, 16)]
                        mr[p, pl.ds(cj, 16)] = q
                        a = (wr[p, pl.ds(CB + cj, 16)]
                             * gr[p, pl.ds(CB + cj, 16)])
                        c = (wr[p, pl.ds(2 * CB + cj, 16)]
                             * gr[p, pl.ds(2 * CB + cj, 16)])
                        for k in range(3):
                            mr[p, pl.ds(CB + k * CB + cj, 16)] = (
                                a * dk[k]
                                + c * gr[p, pl.ds(SEG + k * CB + cj, 16)])
                return 0

            lax.fori_loop(0, WW // 2, _body, 0)

        def scatter(rb):
            pltpu.sync_copy(msg_v[rb], acc_sh.at[idxi_v[rb]], add=True)

        def zero_rows(rb, lo, hi):
            z = jnp.zeros((16,), jnp.float32)
            for r in range(lo, hi):
                for j in range(ROW // 16):
                    msg_v[rb][r, pl.ds(j * 16, 16)] = z

        issue_in(0, 0)
        drain_in(0, 0)
        shift(0)
        issue_g(0)
        issue_in(1, 1)

        def outer(g2, _):
            for rb in (0, 1):
                w = g2 * 2 + rb
                ob = 1 - rb
                drain_in(w + 1, ob)
                shift(ob)
                issue_g(ob)
                drain_g(rb)
                compute(rb)
                scatter(rb)
                issue_in(w + 2, rb)
            return 0

        lax.fori_loop(0, (NWIN - 2) // 2, outer, 0)
        drain_in(NWIN - 1, 1)
        drain_g(0)
        compute(0)
        zero_rows(0, WW - 16, WW)
        scatter(0)
        plsc.subcore_barrier()
        rows_io(True, b)
        plsc.subcore_barrier()


def _assemble_body(o_ref, q_ref, mu_ref):
    o = o_ref[...]
    q = jnp.concatenate([o[b, :, 0:CB] for b in range(NB)], axis=-1)
    q_ref[...] = q[:, None, :]
    mu = jnp.stack(
        [jnp.concatenate([o[b, :, CB + k * CB:CB + (k + 1) * CB]
                          for b in range(NB)], axis=-1)
         for k in range(3)], axis=1)
    mu_ref[...] = mu


def kernel(per_atom_scalar_representation, per_atom_vector_representation,
           W_ij, dir_ij, pairlist, W1, b1, W2, b2):
    s2 = per_atom_scalar_representation.reshape(N, D)
    vflat = per_atom_vector_representation.reshape(N, 3 * D)
    idx_i = pairlist[0]
    idx_j = pairlist[1]

    ng = N // TCB
    g_blk, res_blk = pl.pallas_call(
        _mlp_body,
        grid=(ng,),
        in_specs=[
            pl.BlockSpec((TCB, D), lambda i: (i, 0)),
            pl.BlockSpec((TCB, 3 * D), lambda i: (i, 0)),
            pl.BlockSpec((D, D), lambda i: (0, 0)),
            pl.BlockSpec((D,), lambda i: (0,)),
            pl.BlockSpec((D, 3 * D), lambda i: (0, 0)),
            pl.BlockSpec((3 * D,), lambda i: (0,)),
        ],
        out_specs=[
            pl.BlockSpec((NB, TCB, GROW), lambda i: (0, i, 0)),
            pl.BlockSpec((NB, TCB, ROW), lambda i: (0, i, 0)),
        ],
        out_shape=[
            jax.ShapeDtypeStruct((NB, N, GROW), jnp.float32),
            jax.ShapeDtypeStruct((NB, N, ROW), jnp.float32),
        ],
    )(s2, vflat, W1, b1, W2, b2)

    npg = P // PCB
    w_blk = pl.pallas_call(
        _wblk_body,
        grid=(npg,),
        in_specs=[pl.BlockSpec((PCB, 3 * D), lambda i: (i, 0))],
        out_specs=pl.BlockSpec((NB, PCB, SEG), lambda i: (0, i, 0)),
        out_shape=jax.ShapeDtypeStruct((NB, P_PAD, SEG), jnp.float32),
    )(W_ij)

    gflat = g_blk.reshape(NB * N, GROW)
    resflat = res_blk.reshape(NB * N, ROW)
    wflat = w_blk.reshape(NB * P_PAD, SEG)

    npad = P_PAD - P
    idx_i_p = jnp.concatenate([idx_i, jnp.zeros((npad,), jnp.int32)])
    idx_j_p = jnp.concatenate([idx_j, jnp.zeros((npad,), jnp.int32)])
    dir_p = jnp.concatenate([dir_ij.reshape(P * 3),
                             jnp.zeros((npad * 3,), jnp.float32)])

    mesh = plsc.VectorSubcoreMesh(core_axis_name="c", subcore_axis_name="s")
    sc = pl.kernel(
        _sc_body,
        out_type=jax.ShapeDtypeStruct((NB * N, ROW), jnp.float32),
        mesh=mesh,
        scratch_types=[
            pltpu.VMEM_SHARED((N, ROW), jnp.float32),
            pltpu.VMEM((WW,), jnp.int32),
            pltpu.VMEM((WW,), jnp.int32),
            pltpu.VMEM((WW,), jnp.int32),
            pltpu.VMEM((WW,), jnp.int32),
            pltpu.VMEM((WW,), jnp.int32),
            pltpu.VMEM((WW,), jnp.int32),
            pltpu.VMEM((WW, SEG), jnp.float32),
            pltpu.VMEM((WW, SEG), jnp.float32),
            pltpu.VMEM((WW, GROW), jnp.float32),
            pltpu.VMEM((WW, GROW), jnp.float32),
            pltpu.VMEM((WW * 3 + 16,), jnp.float32),
            pltpu.VMEM((WW * 3 + 16,), jnp.float32),
            pltpu.VMEM((WW, ROW), jnp.float32),
            pltpu.VMEM((WW, ROW), jnp.float32),
            pltpu.SemaphoreType.DMA,
            pltpu.SemaphoreType.DMA,
            pltpu.SemaphoreType.DMA,
            pltpu.SemaphoreType.DMA,
        ],
    )
    out_blk = sc(gflat, resflat, wflat, dir_p, idx_i_p, idx_j_p)

    q, mu = pl.pallas_call(
        _assemble_body,
        grid=(ng,),
        in_specs=[pl.BlockSpec((NB, TCB, ROW), lambda i: (0, i, 0))],
        out_specs=[
            pl.BlockSpec((TCB, 1, D), lambda i: (i, 0, 0)),
            pl.BlockSpec((TCB, 3, D), lambda i: (i, 0, 0)),
        ],
        out_shape=[
            jax.ShapeDtypeStruct((N, 1, D), jnp.float32),
            jax.ShapeDtypeStruct((N, 3, D), jnp.float32),
        ],
    )(out_blk.reshape(NB, N, ROW))

    return (q, mu)


# speedup vs baseline: 1.0622x; 1.0622x over previous
import functools

import jax
import jax.numpy as jnp
from jax import lax
from jax.experimental import pallas as pl
from jax.experimental.pallas import tpu as pltpu
from jax.experimental.pallas import tpu_sc as plsc

N = 10000
P = 320000
D = 128
CB = 32
NB = D // CB
SEG = 3 * CB
ROW = 4 * CB

NTILES = 16
ROWS_A = 632
ROWS_LAST = N - 15 * ROWS_A
PAIRS_PER_TILE = P // NTILES
WW = 48
NWIN = 418
P_PAD = 15 * PAIRS_PER_TILE + NWIN * WW

TCB = 1000
PCB = 2000


GROW = 256


def _mlp_body(s_ref, v_ref, w1_ref, b1_ref, w2_ref, b2_ref,
              g_ref, res_ref):
    x = s_ref[...]
    h = jax.nn.silu(jnp.dot(x, w1_ref[...], preferred_element_type=jnp.float32)
                    + b1_ref[...])
    t = jnp.dot(h, w2_ref[...], preferred_element_type=jnp.float32) + b2_ref[...]
    vv = v_ref[...]
    pad = jnp.zeros((x.shape[0], GROW - 2 * SEG), jnp.float32)
    for b in range(NB):
        c0 = CB * b
        tb = jnp.concatenate(
            [t[:, k * D + c0:k * D + c0 + CB] for k in range(3)], axis=1)
        vb = jnp.concatenate(
            [vv[:, k * D + c0:k * D + c0 + CB] for k in range(3)], axis=1)
        g_ref[b] = jnp.concatenate([tb, vb, pad], axis=1)
        res_ref[b] = jnp.concatenate([x[:, c0:c0 + CB], vb], axis=1)


def _wblk_body(w_ref, wb_ref):
    w = w_ref[...]
    for b in range(NB):
        c0 = CB * b
        wb_ref[b] = jnp.concatenate(
            [w[:, k * D + c0:k * D + c0 + CB] for k in range(3)], axis=1)


def _sc_body(g_hbm, res_hbm, w_hbm, dir_hbm, ii_hbm, ij_hbm,
             out_hbm,
             acc_sh, idxi0, idxi1, idxj0, idxj1, idxs0, idxs1,
             isc0, isc1, w0, w1, g0, g1, dir0, dir1, msg0, msg1,
             semi0, semi1, semg0, semg1, sems0, sems1):
    cid = lax.axis_index("c")
    sid = lax.axis_index("s")
    pair0 = sid * PAIRS_PER_TILE
    idxi_v = (idxi0, idxi1)
    idxj_v = (idxj0, idxj1)
    idxs_v = (idxs0, idxs1)
    w_v = (w0, w1)
    g_v = (g0, g1)
    dir_v = (dir0, dir1)
    msg_v = (msg0, msg1)
    idxsc = (isc0, isc1)
    semi = (semi0, semi1)
    semg = (semg0, semg1)
    sems = (sems0, sems1)

    def rows_io(flush, b):
        def do(row0, nrows):
            src = (acc_sh.at[pl.ds(row0, nrows)] if flush else
                   res_hbm.at[pl.ds(b * N + row0, nrows)])
            dst = (out_hbm.at[pl.ds(b * N + row0, nrows)] if flush else
                   acc_sh.at[pl.ds(row0, nrows)])
            pltpu.sync_copy(src, dst)
        @pl.when(sid < 15)
        def _():
            do(sid * ROWS_A, ROWS_A)
        @pl.when(sid == 15)
        def _():
            do(15 * ROWS_A, ROWS_LAST)

    for bi in range(NB // 2):
        b = cid * (NB // 2) + bi
        rows_io(False, b)
        plsc.subcore_barrier()

        def in_copies(w, rb):
            base = pair0 + w * WW
            return [
                (ii_hbm.at[pl.ds(base, WW)], idxi_v[rb]),
                (ij_hbm.at[pl.ds(base, WW)], idxj_v[rb]),
                (dir_hbm.at[pl.ds(base * 3, WW * 3)],
                 dir_v[rb].at[pl.ds(0, WW * 3)]),
                (w_hbm.at[pl.ds(b * P_PAD + base, WW)], w_v[rb]),
            ]

        def issue_in(w, rb):
            for s, d in in_copies(w, rb):
                pltpu.async_copy(s, d, semi[rb])

        def drain_in(w, rb):
            for s, d in in_copies(w, rb):
                pltpu.make_async_copy(s, d, semi[rb]).wait()

        def shift(rb):
            for i in range(WW // 16):
                idxs_v[rb][pl.ds(i * 16, 16)] = (
                    idxj_v[rb][pl.ds(i * 16, 16)] + b * N)

        def issue_g(rb):
            pltpu.async_copy(g_hbm.at[idxs_v[rb]], g_v[rb], semg[rb])

        def drain_g(rb):
            pltpu.make_async_copy(g_hbm.at[idxs_v[rb]], g_v[rb],
                                  semg[rb]).wait()

        def compute(rb):
            wr, gr, dr, mr = w_v[rb], g_v[rb], dir_v[rb], msg_v[rb]

            def _body(p2, _):
                for u in range(2):
                    p = p2 * 2 + u
                    dvec = dr[pl.ds(p * 3, 16)]
                    dk = [dvec[0], dvec[1], dvec[2]]
                    for j in range(CB // 16):
                        cj = j * 16
                        q = wr[p, pl.ds(cj, 16)] * gr[p, pl.ds(cj, 16)]
                        mr[p, pl.ds(cj, 16)] = q
                        a = (wr[p, pl.ds(CB + cj, 16)]
                             * gr[p, pl.ds(CB + cj, 16)])
                        c = (wr[p, pl.ds(2 * CB + cj, 16)]
                             * gr[p, pl.ds(2 * CB + cj, 16)])
                        for k in range(3):
                            mr[p, pl.ds(CB + k * CB + cj, 16)] = (
                                a * dk[k]
                                + c * gr[p, pl.ds(SEG + k * CB + cj, 16)])
                return 0

            lax.fori_loop(0, WW // 2, _body, 0)

        def idxcopy(rb):
            for i in range(WW // 16):
                idxsc[rb][pl.ds(i * 16, 16)] = idxi_v[rb][pl.ds(i * 16, 16)]

        def issue_sc(rb):
            pltpu.async_copy(msg_v[rb], acc_sh.at[idxsc[rb]], sems[rb],
                             add=True)

        def drain_sc(rb):
            pltpu.make_async_copy(msg_v[rb], acc_sh.at[idxsc[rb]],
                                  sems[rb]).wait()

        def zero_rows(rb, lo, hi):
            z = jnp.zeros((16,), jnp.float32)
            for r in range(lo, hi):
                for j in range(ROW // 16):
                    msg_v[rb][r, pl.ds(j * 16, 16)] = z

        issue_in(0, 0)
        drain_in(0, 0)
        shift(0)
        issue_g(0)
        issue_in(1, 1)
        for w01 in (0, 1):
            rb, ob = w01 % 2, 1 - w01 % 2
            drain_in(w01 + 1, ob)
            shift(ob)
            issue_g(ob)
            drain_g(rb)
            idxcopy(rb)
            compute(rb)
            issue_sc(rb)
            issue_in(w01 + 2, rb)

        def outer(g2, _):
            for rb in (0, 1):
                w = g2 * 2 + rb
                ob = 1 - rb
                drain_in(w + 1, ob)
                shift(ob)
                issue_g(ob)
                drain_g(rb)
                drain_sc(rb)
                idxcopy(rb)
                compute(rb)
                issue_sc(rb)
                issue_in(w + 2, rb)
            return 0

        lax.fori_loop(1, (NWIN - 2) // 2, outer, 0)
        drain_in(NWIN - 1, 1)
        drain_g(0)
        drain_sc(0)
        idxcopy(0)
        compute(0)
        zero_rows(0, WW - 16, WW)
        issue_sc(0)
        drain_sc(1)
        drain_sc(0)
        plsc.subcore_barrier()
        rows_io(True, b)
        plsc.subcore_barrier()


def _assemble_body(o_ref, q_ref, mu_ref):
    o = o_ref[...]
    q = jnp.concatenate([o[b, :, 0:CB] for b in range(NB)], axis=-1)
    q_ref[...] = q[:, None, :]
    mu = jnp.stack(
        [jnp.concatenate([o[b, :, CB + k * CB:CB + (k + 1) * CB]
                          for b in range(NB)], axis=-1)
         for k in range(3)], axis=1)
    mu_ref[...] = mu


def kernel(per_atom_scalar_representation, per_atom_vector_representation,
           W_ij, dir_ij, pairlist, W1, b1, W2, b2):
    s2 = per_atom_scalar_representation.reshape(N, D)
    vflat = per_atom_vector_representation.reshape(N, 3 * D)
    idx_i = pairlist[0]
    idx_j = pairlist[1]

    ng = N // TCB
    g_blk, res_blk = pl.pallas_call(
        _mlp_body,
        grid=(ng,),
        in_specs=[
            pl.BlockSpec((TCB, D), lambda i: (i, 0)),
            pl.BlockSpec((TCB, 3 * D), lambda i: (i, 0)),
            pl.BlockSpec((D, D), lambda i: (0, 0)),
            pl.BlockSpec((D,), lambda i: (0,)),
            pl.BlockSpec((D, 3 * D), lambda i: (0, 0)),
            pl.BlockSpec((3 * D,), lambda i: (0,)),
        ],
        out_specs=[
            pl.BlockSpec((NB, TCB, GROW), lambda i: (0, i, 0)),
            pl.BlockSpec((NB, TCB, ROW), lambda i: (0, i, 0)),
        ],
        out_shape=[
            jax.ShapeDtypeStruct((NB, N, GROW), jnp.float32),
            jax.ShapeDtypeStruct((NB, N, ROW), jnp.float32),
        ],
    )(s2, vflat, W1, b1, W2, b2)

    npg = P // PCB
    w_blk = pl.pallas_call(
        _wblk_body,
        grid=(npg,),
        in_specs=[pl.BlockSpec((PCB, 3 * D), lambda i: (i, 0))],
        out_specs=pl.BlockSpec((NB, PCB, SEG), lambda i: (0, i, 0)),
        out_shape=jax.ShapeDtypeStruct((NB, P_PAD, SEG), jnp.float32),
    )(W_ij)

    gflat = g_blk.reshape(NB * N, GROW)
    resflat = res_blk.reshape(NB * N, ROW)
    wflat = w_blk.reshape(NB * P_PAD, SEG)

    npad = P_PAD - P
    idx_i_p = jnp.concatenate([idx_i, jnp.zeros((npad,), jnp.int32)])
    idx_j_p = jnp.concatenate([idx_j, jnp.zeros((npad,), jnp.int32)])
    dir_p = jnp.concatenate([dir_ij.reshape(P * 3),
                             jnp.zeros((npad * 3,), jnp.float32)])

    mesh = plsc.VectorSubcoreMesh(core_axis_name="c", subcore_axis_name="s")
    sc = pl.kernel(
        _sc_body,
        out_type=jax.ShapeDtypeStruct((NB * N, ROW), jnp.float32),
        mesh=mesh,
        scratch_types=[
            pltpu.VMEM_SHARED((N, ROW), jnp.float32),
            pltpu.VMEM((WW,), jnp.int32),
            pltpu.VMEM((WW,), jnp.int32),
            pltpu.VMEM((WW,), jnp.int32),
            pltpu.VMEM((WW,), jnp.int32),
            pltpu.VMEM((WW,), jnp.int32),
            pltpu.VMEM((WW,), jnp.int32),
            pltpu.VMEM((WW,), jnp.int32),
            pltpu.VMEM((WW,), jnp.int32),
            pltpu.VMEM((WW, SEG), jnp.float32),
            pltpu.VMEM((WW, SEG), jnp.float32),
            pltpu.VMEM((WW, GROW), jnp.float32),
            pltpu.VMEM((WW, GROW), jnp.float32),
            pltpu.VMEM((WW * 3 + 16,), jnp.float32),
            pltpu.VMEM((WW * 3 + 16,), jnp.float32),
            pltpu.VMEM((WW, ROW), jnp.float32),
            pltpu.VMEM((WW, ROW), jnp.float32),
            pltpu.SemaphoreType.DMA,
            pltpu.SemaphoreType.DMA,
            pltpu.SemaphoreType.DMA,
            pltpu.SemaphoreType.DMA,
            pltpu.SemaphoreType.DMA,
            pltpu.SemaphoreType.DMA,
        ],
    )
    out_blk = sc(gflat, resflat, wflat, dir_p, idx_i_p, idx_j_p)

    q, mu = pl.pallas_call(
        _assemble_body,
        grid=(ng,),
        in_specs=[pl.BlockSpec((NB, TCB, ROW), lambda i: (0, i, 0))],
        out_specs=[
            pl.BlockSpec((TCB, 1, D), lambda i: (i, 0, 0)),
            pl.BlockSpec((TCB, 3, D), lambda i: (i, 0, 0)),
        ],
        out_shape=[
            jax.ShapeDtypeStruct((N, 1, D), jnp.float32),
            jax.ShapeDtypeStruct((N, 3, D), jnp.float32),
        ],
    )(out_blk.reshape(NB, N, ROW))

    return (q, mu)


# speedup vs baseline: 1.0623x; 1.0001x over previous
import jax
import jax.numpy as jnp
from jax import lax
from jax.experimental import pallas as pl
from jax.experimental.pallas import tpu as pltpu
from jax.experimental.pallas import tpu_sc as plsc

N = 10000
P = 320000
D = 128
CB = 32
NB = D // CB
SEG = 3 * CB
ROW = 4 * CB

NTILES = 16
ROWS_A = 632
ROWS_LAST = N - 15 * ROWS_A
PAIRS_PER_TILE = P // NTILES
WW = 48
NWIN = 418
P_PAD = 15 * PAIRS_PER_TILE + NWIN * WW

TCB = 1000
PCB = 2000


GROW = 256


def _mlp_body(s_ref, v_ref, w1_ref, b1_ref, w2_ref, b2_ref,
              g_ref, res_ref):
    x = s_ref[...]
    h = jax.nn.silu(jnp.dot(x, w1_ref[...], preferred_element_type=jnp.float32)
                    + b1_ref[...])
    t = jnp.dot(h, w2_ref[...], preferred_element_type=jnp.float32) + b2_ref[...]
    vv = v_ref[...]
    pad = jnp.zeros((x.shape[0], GROW - 2 * SEG), jnp.float32)
    for b in range(NB):
        c0 = CB * b
        tb = jnp.concatenate(
            [t[:, k * D + c0:k * D + c0 + CB] for k in range(3)], axis=1)
        vb = jnp.concatenate(
            [vv[:, k * D + c0:k * D + c0 + CB] for k in range(3)], axis=1)
        g_ref[b] = jnp.concatenate([tb, vb, pad], axis=1)
        res_ref[b] = jnp.concatenate([x[:, c0:c0 + CB], vb], axis=1)


def _wblk_body(w_ref, wb_ref):
    w = w_ref[...]
    for b in range(NB):
        c0 = CB * b
        wb_ref[b] = jnp.concatenate(
            [w[:, k * D + c0:k * D + c0 + CB] for k in range(3)], axis=1)


def _sc_body(g_hbm, res_hbm, w_hbm, dir_hbm, ii_hbm, ij_hbm,
             out_hbm,
             acc_sh, idxi0, idxi1, idxj0, idxj1, idxs0, idxs1,
             isc0, isc1, w0, w1, g0, g1, dir0, dir1, msg0, msg1,
             semi0, semi1, semg0, semg1, sems0, sems1):
    cid = lax.axis_index("c")
    sid = lax.axis_index("s")
    pair0 = sid * PAIRS_PER_TILE
    idxi_v = (idxi0, idxi1)
    idxj_v = (idxj0, idxj1)
    idxs_v = (idxs0, idxs1)
    w_v = (w0, w1)
    g_v = (g0, g1)
    dir_v = (dir0, dir1)
    msg_v = (msg0, msg1)
    idxsc = (isc0, isc1)
    semi = (semi0, semi1)
    semg = (semg0, semg1)
    sems = (sems0, sems1)

    def rows_io(flush, b):
        def do(row0, nrows):
            src = (acc_sh.at[pl.ds(row0, nrows)] if flush else
                   res_hbm.at[pl.ds(b * N + row0, nrows)])
            dst = (out_hbm.at[pl.ds(b * N + row0, nrows)] if flush else
                   acc_sh.at[pl.ds(row0, nrows)])
            pltpu.sync_copy(src, dst)
        @pl.when(sid < 15)
        def _():
            do(sid * ROWS_A, ROWS_A)
        @pl.when(sid == 15)
        def _():
            do(15 * ROWS_A, ROWS_LAST)

    for bi in range(NB // 2):
        b = cid * (NB // 2) + bi
        rows_io(False, b)
        plsc.subcore_barrier()

        def in_copies(w, rb):
            base = pair0 + w * WW
            return [
                (ii_hbm.at[pl.ds(base, WW)], idxi_v[rb]),
                (ij_hbm.at[pl.ds(base, WW)], idxj_v[rb]),
                (dir_hbm.at[pl.ds(base * 3, WW * 3)],
                 dir_v[rb].at[pl.ds(0, WW * 3)]),
                (w_hbm.at[pl.ds(b * P_PAD + base, WW)], w_v[rb]),
            ]

        def issue_in(w, rb):
            for s, d in in_copies(w, rb):
                pltpu.async_copy(s, d, semi[rb])

        def drain_in(w, rb):
            for s, d in in_copies(w, rb):
                pltpu.make_async_copy(s, d, semi[rb]).wait()

        def shift(rb):
            for i in range(WW // 16):
                idxs_v[rb][pl.ds(i * 16, 16)] = (
                    idxj_v[rb][pl.ds(i * 16, 16)] + b * N)

        def issue_g(rb):
            pltpu.async_copy(g_hbm.at[idxs_v[rb]], g_v[rb], semg[rb])

        def drain_g(rb):
            pltpu.make_async_copy(g_hbm.at[idxs_v[rb]], g_v[rb],
                                  semg[rb]).wait()

        def compute(rb):
            wr, gr, dr, mr = w_v[rb], g_v[rb], dir_v[rb], msg_v[rb]

            def _body(p2, _):
                for u in range(2):
                    p = p2 * 2 + u
                    dvec = dr[pl.ds(p * 3, 16)]
                    dk = [dvec[0], dvec[1], dvec[2]]
                    for j in range(CB // 16):
                        cj = j * 16
                        q = wr[p, pl.ds(cj, 16)] * gr[p, pl.ds(cj, 16)]
                        mr[p, pl.ds(cj, 16)] = q
                        a = (wr[p, pl.ds(CB + cj, 16)]
                             * gr[p, pl.ds(CB + cj, 16)])
                        c = (wr[p, pl.ds(2 * CB + cj, 16)]
                             * gr[p, pl.ds(2 * CB + cj, 16)])
                        for k in range(3):
                            mr[p, pl.ds(CB + k * CB + cj, 16)] = (
                                a * dk[k]
                                + c * gr[p, pl.ds(SEG + k * CB + cj, 16)])
                return 0

            lax.fori_loop(0, WW // 2, _body, 0)

        def idxcopy(rb):
            for i in range(WW // 16):
                idxsc[rb][pl.ds(i * 16, 16)] = idxi_v[rb][pl.ds(i * 16, 16)]

        def issue_sc(rb):
            pltpu.async_copy(msg_v[rb], acc_sh.at[idxsc[rb]], sems[rb],
                             add=True)

        def drain_sc(rb):
            pltpu.make_async_copy(msg_v[rb], acc_sh.at[idxsc[rb]],
                                  sems[rb]).wait()

        def zero_rows(rb, lo, hi):
            z = jnp.zeros((16,), jnp.float32)
            for r in range(lo, hi):
                for j in range(ROW // 16):
                    msg_v[rb][r, pl.ds(j * 16, 16)] = z

        issue_in(0, 0)
        drain_in(0, 0)
        shift(0)
        issue_g(0)
        issue_in(1, 1)
        for w01 in (0, 1):
            rb, ob = w01 % 2, 1 - w01 % 2
            drain_in(w01 + 1, ob)
            shift(ob)
            issue_g(ob)
            drain_g(rb)
            idxcopy(rb)
            compute(rb)
            issue_sc(rb)
            issue_in(w01 + 2, rb)

        def outer(g2, _):
            for rb in (0, 1):
                w = g2 * 2 + rb
                ob = 1 - rb
                drain_in(w + 1, ob)
                shift(ob)
                issue_g(ob)
                drain_g(rb)
                drain_sc(rb)
                idxcopy(rb)
                compute(rb)
                issue_sc(rb)
                issue_in(w + 2, rb)
            return 0

        lax.fori_loop(1, (NWIN - 2) // 2, outer, 0)
        drain_in(NWIN - 1, 1)
        drain_g(0)
        drain_sc(0)
        idxcopy(0)
        compute(0)
        zero_rows(0, WW - 16, WW)
        issue_sc(0)
        drain_sc(1)
        drain_sc(0)
        plsc.subcore_barrier()
        rows_io(True, b)
        plsc.subcore_barrier()


def _assemble_body(o_ref, q_ref, mu_ref):
    o = o_ref[...]
    q = jnp.concatenate([o[b, :, 0:CB] for b in range(NB)], axis=-1)
    q_ref[...] = q[:, None, :]
    mu = jnp.stack(
        [jnp.concatenate([o[b, :, CB + k * CB:CB + (k + 1) * CB]
                          for b in range(NB)], axis=-1)
         for k in range(3)], axis=1)
    mu_ref[...] = mu


def kernel(per_atom_scalar_representation, per_atom_vector_representation,
           W_ij, dir_ij, pairlist, W1, b1, W2, b2):
    s2 = per_atom_scalar_representation.reshape(N, D)
    vflat = per_atom_vector_representation.reshape(N, 3 * D)
    idx_i = pairlist[0]
    idx_j = pairlist[1]

    ng = N // TCB
    g_blk, res_blk = pl.pallas_call(
        _mlp_body,
        grid=(ng,),
        in_specs=[
            pl.BlockSpec((TCB, D), lambda i: (i, 0)),
            pl.BlockSpec((TCB, 3 * D), lambda i: (i, 0)),
            pl.BlockSpec((D, D), lambda i: (0, 0)),
            pl.BlockSpec((D,), lambda i: (0,)),
            pl.BlockSpec((D, 3 * D), lambda i: (0, 0)),
            pl.BlockSpec((3 * D,), lambda i: (0,)),
        ],
        out_specs=[
            pl.BlockSpec((NB, TCB, GROW), lambda i: (0, i, 0)),
            pl.BlockSpec((NB, TCB, ROW), lambda i: (0, i, 0)),
        ],
        out_shape=[
            jax.ShapeDtypeStruct((NB, N, GROW), jnp.float32),
            jax.ShapeDtypeStruct((NB, N, ROW), jnp.float32),
        ],
    )(s2, vflat, W1, b1, W2, b2)

    npg = P // PCB
    w_blk = pl.pallas_call(
        _wblk_body,
        grid=(npg,),
        in_specs=[pl.BlockSpec((PCB, 3 * D), lambda i: (i, 0))],
        out_specs=pl.BlockSpec((NB, PCB, SEG), lambda i: (0, i, 0)),
        out_shape=jax.ShapeDtypeStruct((NB, P_PAD, SEG), jnp.float32),
    )(W_ij)

    gflat = g_blk.reshape(NB * N, GROW)
    resflat = res_blk.reshape(NB * N, ROW)
    wflat = w_blk.reshape(NB * P_PAD, SEG)

    npad = P_PAD - P
    idx_i_p = jnp.concatenate([idx_i, jnp.zeros((npad,), jnp.int32)])
    idx_j_p = jnp.concatenate([idx_j, jnp.zeros((npad,), jnp.int32)])
    dir_p = jnp.concatenate([dir_ij.reshape(P * 3),
                             jnp.zeros((npad * 3,), jnp.float32)])

    mesh = plsc.VectorSubcoreMesh(core_axis_name="c", subcore_axis_name="s")
    sc = pl.kernel(
        _sc_body,
        out_type=jax.ShapeDtypeStruct((NB * N, ROW), jnp.float32),
        mesh=mesh,
        scratch_types=[
            pltpu.VMEM_SHARED((N, ROW), jnp.float32),
            pltpu.VMEM((WW,), jnp.int32),
            pltpu.VMEM((WW,), jnp.int32),
            pltpu.VMEM((WW,), jnp.int32),
            pltpu.VMEM((WW,), jnp.int32),
            pltpu.VMEM((WW,), jnp.int32),
            pltpu.VMEM((WW,), jnp.int32),
            pltpu.VMEM((WW,), jnp.int32),
            pltpu.VMEM((WW,), jnp.int32),
            pltpu.VMEM((WW, SEG), jnp.float32),
            pltpu.VMEM((WW, SEG), jnp.float32),
            pltpu.VMEM((WW, GROW), jnp.float32),
            pltpu.VMEM((WW, GROW), jnp.float32),
            pltpu.VMEM((WW * 3 + 16,), jnp.float32),
            pltpu.VMEM((WW * 3 + 16,), jnp.float32),
            pltpu.VMEM((WW, ROW), jnp.float32),
            pltpu.VMEM((WW, ROW), jnp.float32),
            pltpu.SemaphoreType.DMA,
            pltpu.SemaphoreType.DMA,
            pltpu.SemaphoreType.DMA,
            pltpu.SemaphoreType.DMA,
            pltpu.SemaphoreType.DMA,
            pltpu.SemaphoreType.DMA,
        ],
    )
    out_blk = sc(gflat, resflat, wflat, dir_p, idx_i_p, idx_j_p)

    q, mu = pl.pallas_call(
        _assemble_body,
        grid=(ng,),
        in_specs=[pl.BlockSpec((NB, TCB, ROW), lambda i: (0, i, 0))],
        out_specs=[
            pl.BlockSpec((TCB, 1, D), lambda i: (i, 0, 0)),
            pl.BlockSpec((TCB, 3, D), lambda i: (i, 0, 0)),
        ],
        out_shape=[
            jax.ShapeDtypeStruct((N, 1, D), jnp.float32),
            jax.ShapeDtypeStruct((N, 3, D), jnp.float32),
        ],
    )(out_blk.reshape(NB, N, ROW))

    return (q, mu)
